# baseline (speedup 1.0000x reference)
import functools

import jax
import jax.numpy as jnp
from jax import lax
from jax.experimental import pallas as pl
from jax.experimental.pallas import tpu as pltpu
from jax.experimental.pallas import tpu_sc as plsc

_N = 10000
_H = 8
_NW = 32
_NPT = 320
_NPAD = _NW * _NPT


def _mm_attn_body(x_ref, w_ref, al_ref, ar_ref, featx_ref, er_ref, *, heads, d_out):
    feat = jnp.dot(x_ref[...], w_ref[...], preferred_element_type=jnp.float32)
    b = feat.shape[0]
    f3 = feat.reshape(b, heads, d_out)
    el = jnp.sum(f3 * al_ref[...], axis=-1)
    er = jnp.sum(f3 * ar_ref[...], axis=-1)
    featx_ref[...] = jnp.concatenate(
        [feat, el, jnp.zeros((b, 128 - heads), jnp.float32)], axis=1)
    er_ref[...] = jnp.concatenate(
        [er, jnp.zeros((b, 16 - heads), jnp.float32)], axis=1)


def _mm_attn(x, w, al, ar, heads, d_out, block_rows):
    n, d_in = x.shape
    hd = heads * d_out
    kernel = functools.partial(_mm_attn_body, heads=heads, d_out=d_out)
    return pl.pallas_call(
        kernel,
        grid=(n // block_rows,),
        in_specs=[
            pl.BlockSpec((block_rows, d_in), lambda i: (i, 0)),
            pl.BlockSpec((d_in, hd), lambda i: (0, 0)),
            pl.BlockSpec((1, heads, d_out), lambda i: (0, 0, 0)),
            pl.BlockSpec((1, heads, d_out), lambda i: (0, 0, 0)),
        ],
        out_specs=[
            pl.BlockSpec((block_rows, hd + 128), lambda i: (i, 0)),
            pl.BlockSpec((block_rows, 16), lambda i: (i, 0)),
        ],
        out_shape=[
            jax.ShapeDtypeStruct((n, hd + 128), jnp.float32),
            jax.ShapeDtypeStruct((n, 16), jnp.float32),
        ],
    )(x, w, al, ar)


def _gat_edge_sc(featx, er16, ssrc, offs, bias, d_head, mean_heads):
    hd = _H * d_head
    out_cols = d_head if mean_heads else hd
    nj = d_head // 16
    mesh = plsc.VectorSubcoreMesh(core_axis_name="c", subcore_axis_name="s")

    @functools.partial(
        pl.kernel,
        out_type=jax.ShapeDtypeStruct((_NPAD, out_cols), jnp.float32),
        mesh=mesh,
        scratch_types=[
            pltpu.VMEM((_NPT + 24,), jnp.int32),
            pltpu.VMEM((_NPT, 16), jnp.float32),
            pltpu.VMEM((32,), jnp.int32),
            pltpu.VMEM((32, hd + 128), jnp.float32),
            pltpu.VMEM((16,), jnp.float32),
            pltpu.VMEM((_H, d_head), jnp.float32),
            pltpu.VMEM((out_cols,), jnp.float32),
            pltpu.VMEM((hd,), jnp.float32),
            pltpu.SemaphoreType.DMA,
            pltpu.SemaphoreType.DMA,
        ],
    )
    def k(featx_h, er_h, ssrc_h, offs_h, bias_h, out_h,
          offs_v, er_v, idx_v, rows_v, den_v, acc_v, orow_v, bias_v,
          sem_a, sem_b):
        wid = lax.axis_index("s") * 2 + lax.axis_index("c")
        n0 = wid * _NPT
        pltpu.sync_copy(offs_h.at[pl.ds(n0, _NPT + 24)], offs_v)
        pltpu.sync_copy(er_h.at[pl.ds(n0, _NPT)], er_v)
        pltpu.sync_copy(bias_h, bias_v)

        def node_body(dl, _):
            d = n0 + dl

            @pl.when(d < _N)
            def _():
                ovec = offs_v[pl.ds(dl, 16)]
                s = ovec[0]
                e_end = ovec[1]
                s8 = (s // 8) * 8
                nch = (e_end - s8 + 15) // 16
                den_v[...] = jnp.zeros((16,), jnp.float32)
                for h in range(_H):
                    for j in range(nj):
                        acc_v[h, pl.ds(16 * j, 16)] = jnp.zeros((16,), jnp.float32)
                er_row = er_v[dl]

                def issue(c, slot, sem):
                    base = s8 + c * 16
                    pltpu.sync_copy(ssrc_h.at[pl.ds(base, 16)],
                                    idx_v.at[pl.ds(slot * 16, 16)])
                    idxvec = idx_v[pl.ds(slot * 16, 16)]
                    pltpu.async_copy(
                        featx_h.at[idxvec],
                        rows_v.at[pl.ds(slot * 16, 16)], sem)

                def wait(slot, sem):
                    idxvec = idx_v[pl.ds(slot * 16, 16)]
                    pltpu.make_async_copy(
                        featx_h.at[idxvec],
                        rows_v.at[pl.ds(slot * 16, 16)], sem).wait()

                @pl.when(nch > 0)
                def _():
                    issue(0, 0, sem_a)

                def chunk(c, _c):
                    base = s8 + c * 16
                    even = lax.rem(c, 2) == 0

                    @pl.when((c + 1 < nch) & even)
                    def _():
                        issue(c + 1, 1, sem_b)

                    @pl.when((c + 1 < nch) & jnp.logical_not(even))
                    def _():
                        issue(c + 1, 0, sem_a)

                    @pl.when(even)
                    def _():
                        wait(0, sem_a)

                    @pl.when(jnp.logical_not(even))
                    def _():
                        wait(1, sem_b)

                    ro = lax.rem(c, 2) * 16
                    ees = []
                    for i in range(16):
                        ev = rows_v[ro + i, pl.ds(hd, 16)] + er_row
                        ev = jnp.where(ev > 0, ev, 0.2 * ev)
                        pos = base + i
                        validf = ((pos >= s) & (pos < e_end)).astype(jnp.float32)
                        ees.append(jnp.exp(ev) * jnp.full((16,), validf))
                    tot = ees[0]
                    for i in range(1, 16):
                        tot = tot + ees[i]
                    den_v[...] = den_v[...] + tot
                    for i in range(16):
                        for h in range(_H):
                            wv = jnp.full((16,), ees[i][h], jnp.float32)
                            for j in range(nj):
                                col = h * d_head + 16 * j
                                plsc.addupdate(
                                    acc_v.at[h, pl.ds(16 * j, 16)],
                                    wv * rows_v[ro + i, pl.ds(col, 16)])
                    return 0

                lax.fori_loop(0, nch, chunk, 0)

                den = den_v[...]
                inv_vec = jnp.where(den > 0.0, 1.0 / den, 0.0)
                for h in range(_H):
                    invv = jnp.full((16,), inv_vec[h], jnp.float32)
                    for j in range(nj):
                        col = h * d_head + 16 * j
                        o = acc_v[h, pl.ds(16 * j, 16)] * invv + bias_v[pl.ds(col, 16)]
                        o = jnp.maximum(o, 0.0)
                        if mean_heads:
                            if h == 0:
                                orow_v[pl.ds(16 * j, 16)] = o * 0.125
                            else:
                                plsc.addupdate(orow_v.at[pl.ds(16 * j, 16)], o * 0.125)
                        else:
                            orow_v[pl.ds(col, 16)] = o
                pltpu.sync_copy(orow_v, out_h.at[d])
            return 0

        lax.fori_loop(0, _NPT, node_body, 0)

    return k(featx, er16, ssrc, offs, bias)


def _edge_phase_jnp(feat_flat, el16, er16, src, dst, heads, d_out):
    n = feat_flat.shape[0]
    e = el16[src, :heads] + er16[dst, :heads]
    e = jnp.where(e > 0, e, 0.2 * e)
    ee = jnp.exp(e)
    denom = jax.ops.segment_sum(ee, dst, num_segments=n)
    feat = feat_flat.reshape(n, heads, d_out)
    msg = feat[src] * ee[:, :, None]
    acc = jax.ops.segment_sum(msg, dst, num_segments=n)
    safe = jnp.where(denom > 0, denom, 1.0)
    return acc / safe[:, :, None]


def kernel(feature, edge_index, W1, b1, al1, ar1, W2, b2, al2, ar2):
    src = edge_index[0]
    dst = edge_index[1]

    dst_sorted, src_sorted = lax.sort((dst, src), num_keys=1)
    counts = jax.ops.segment_sum(
        jnp.ones_like(dst), dst, num_segments=_NPAD + 24)
    offs = jnp.concatenate(
        [jnp.zeros((1,), jnp.int32),
         jnp.cumsum(counts).astype(jnp.int32)])[:_NPAD + 24]
    ssrc = jnp.concatenate([src_sorted, jnp.zeros((16,), jnp.int32)])

    x = jnp.pad(feature, ((0, _NPAD - _N), (0, 0)))

    featx1, er1 = _mm_attn(x, W1, al1, ar1, _H, 128, block_rows=512)
    h1 = _gat_edge_sc(featx1, er1, ssrc, offs, b1, 128, mean_heads=False)

    featx2, er2 = _mm_attn(h1, W2, al2, ar2, _H, 64, block_rows=512)
    out = _gat_edge_sc(featx2, er2, ssrc, offs, b2, 64, mean_heads=True)
    return out[:_N]


# speedup vs baseline: 1.8675x; 1.8675x over previous
import functools

import jax
import jax.numpy as jnp
from jax import lax
from jax.experimental import pallas as pl
from jax.experimental.pallas import tpu as pltpu
from jax.experimental.pallas import tpu_sc as plsc

_N = 10000
_H = 8
_NW = 32
_NPT = 320
_NPAD = _NW * _NPT


def _mm_attn_body(x_ref, w_ref, al_ref, ar_ref, featx_ref, er_ref, *, heads, d_out):
    feat = jnp.dot(x_ref[...], w_ref[...], preferred_element_type=jnp.float32)
    b = feat.shape[0]
    f3 = feat.reshape(b, heads, d_out)
    el = jnp.sum(f3 * al_ref[...], axis=-1)
    er = jnp.sum(f3 * ar_ref[...], axis=-1)
    featx_ref[...] = jnp.concatenate(
        [feat, el, jnp.zeros((b, 128 - heads), jnp.float32)], axis=1)
    er_ref[...] = jnp.concatenate(
        [er, jnp.zeros((b, 16 - heads), jnp.float32)], axis=1)


def _mm_attn(x, w, al, ar, heads, d_out, block_rows):
    n, d_in = x.shape
    hd = heads * d_out
    kernel = functools.partial(_mm_attn_body, heads=heads, d_out=d_out)
    return pl.pallas_call(
        kernel,
        grid=(n // block_rows,),
        in_specs=[
            pl.BlockSpec((block_rows, d_in), lambda i: (i, 0)),
            pl.BlockSpec((d_in, hd), lambda i: (0, 0)),
            pl.BlockSpec((1, heads, d_out), lambda i: (0, 0, 0)),
            pl.BlockSpec((1, heads, d_out), lambda i: (0, 0, 0)),
        ],
        out_specs=[
            pl.BlockSpec((block_rows, hd + 128), lambda i: (i, 0)),
            pl.BlockSpec((block_rows, 16), lambda i: (i, 0)),
        ],
        out_shape=[
            jax.ShapeDtypeStruct((n, hd + 128), jnp.float32),
            jax.ShapeDtypeStruct((n, 16), jnp.float32),
        ],
    )(x, w, al, ar)


def _gat_edge_sc(featx, er16, ssrc, offs, bias, d_head, mean_heads):
    hd = _H * d_head
    out_cols = d_head if mean_heads else hd
    nj = d_head // 16
    mesh = plsc.VectorSubcoreMesh(core_axis_name="c", subcore_axis_name="s")

    @functools.partial(
        pl.kernel,
        out_type=jax.ShapeDtypeStruct((_NPAD, out_cols), jnp.float32),
        mesh=mesh,
        scratch_types=[
            pltpu.VMEM((_NPT + 24,), jnp.int32),
            pltpu.VMEM((_NPT, 16), jnp.float32),
            pltpu.VMEM((32,), jnp.int32),
            pltpu.VMEM((32, hd + 128), jnp.float32),
            pltpu.VMEM((16, 16), jnp.float32),
            pltpu.VMEM((16,), jnp.float32),
            pltpu.VMEM((_H, d_head), jnp.float32),
            pltpu.VMEM((out_cols,), jnp.float32),
            pltpu.VMEM((hd,), jnp.float32),
            pltpu.SemaphoreType.DMA,
            pltpu.SemaphoreType.DMA,
        ],
    )
    def k(featx_h, er_h, ssrc_h, offs_h, bias_h, out_h,
          offs_v, er_v, idx_v, rows_v, eev, den_v, acc_v, orow_v, bias_v,
          sem_a, sem_b):
        wid = lax.axis_index("s") * 2 + lax.axis_index("c")
        n0 = wid * _NPT
        pltpu.sync_copy(offs_h.at[pl.ds(n0, _NPT + 24)], offs_v)
        pltpu.sync_copy(er_h.at[pl.ds(n0, _NPT)], er_v)
        pltpu.sync_copy(bias_h, bias_v)

        def node_body(dl, _):
            d = n0 + dl

            @pl.when(d < _N)
            def _():
                ovec = offs_v[pl.ds(dl, 16)]
                s = ovec[0]
                e_end = ovec[1]
                s8 = (s // 8) * 8
                nch = (e_end - s8 + 15) // 16
                den_v[...] = jnp.zeros((16,), jnp.float32)
                for h in range(_H):
                    for j in range(nj):
                        acc_v[h, pl.ds(16 * j, 16)] = jnp.zeros((16,), jnp.float32)
                er_row = er_v[dl]

                def issue(c, slot, sem):
                    base = s8 + c * 16
                    pltpu.sync_copy(ssrc_h.at[pl.ds(base, 16)],
                                    idx_v.at[pl.ds(slot * 16, 16)])
                    idxvec = idx_v[pl.ds(slot * 16, 16)]
                    pltpu.async_copy(
                        featx_h.at[idxvec],
                        rows_v.at[pl.ds(slot * 16, 16)], sem)

                def wait(slot, sem):
                    idxvec = idx_v[pl.ds(slot * 16, 16)]
                    pltpu.make_async_copy(
                        featx_h.at[idxvec],
                        rows_v.at[pl.ds(slot * 16, 16)], sem).wait()

                @pl.when(nch > 0)
                def _():
                    issue(0, 0, sem_a)

                def chunk(c, _c):
                    base = s8 + c * 16
                    even = lax.rem(c, 2) == 0

                    @pl.when((c + 1 < nch) & even)
                    def _():
                        issue(c + 1, 1, sem_b)

                    @pl.when((c + 1 < nch) & jnp.logical_not(even))
                    def _():
                        issue(c + 1, 0, sem_a)

                    @pl.when(even)
                    def _():
                        wait(0, sem_a)

                    @pl.when(jnp.logical_not(even))
                    def _():
                        wait(1, sem_b)

                    ro = lax.rem(c, 2) * 16
                    for i in range(16):
                        ev = rows_v[ro + i, pl.ds(hd, 16)] + er_row
                        ev = jnp.where(ev > 0, ev, 0.2 * ev)
                        eev[i] = jnp.exp(ev)

                    def edge(i, _e):
                        ee_row = eev[i]
                        den_v[...] = den_v[...] + ee_row
                        for h in range(_H):
                            wv = jnp.full((16,), ee_row[h], jnp.float32)
                            for j in range(nj):
                                col = h * d_head + 16 * j
                                plsc.addupdate(
                                    acc_v.at[h, pl.ds(16 * j, 16)],
                                    wv * rows_v[ro + i, pl.ds(col, 16)])
                        return 0

                    lo = jnp.maximum(s - base, 0)
                    hi = jnp.minimum(e_end - base, 16)
                    lax.fori_loop(lo, hi, edge, 0)
                    return 0

                lax.fori_loop(0, nch, chunk, 0)

                den = den_v[...]
                inv_vec = jnp.where(den > 0.0, 1.0 / den, 0.0)
                for h in range(_H):
                    invv = jnp.full((16,), inv_vec[h], jnp.float32)
                    for j in range(nj):
                        col = h * d_head + 16 * j
                        o = acc_v[h, pl.ds(16 * j, 16)] * invv + bias_v[pl.ds(col, 16)]
                        o = jnp.maximum(o, 0.0)
                        if mean_heads:
                            if h == 0:
                                orow_v[pl.ds(16 * j, 16)] = o * 0.125
                            else:
                                plsc.addupdate(orow_v.at[pl.ds(16 * j, 16)], o * 0.125)
                        else:
                            orow_v[pl.ds(col, 16)] = o
                pltpu.sync_copy(orow_v, out_h.at[d])
            return 0

        lax.fori_loop(0, _NPT, node_body, 0)

    return k(featx, er16, ssrc, offs, bias)


def _edge_phase_jnp(feat_flat, el16, er16, src, dst, heads, d_out):
    n = feat_flat.shape[0]
    e = el16[src, :heads] + er16[dst, :heads]
    e = jnp.where(e > 0, e, 0.2 * e)
    ee = jnp.exp(e)
    denom = jax.ops.segment_sum(ee, dst, num_segments=n)
    feat = feat_flat.reshape(n, heads, d_out)
    msg = feat[src] * ee[:, :, None]
    acc = jax.ops.segment_sum(msg, dst, num_segments=n)
    safe = jnp.where(denom > 0, denom, 1.0)
    return acc / safe[:, :, None]


def kernel(feature, edge_index, W1, b1, al1, ar1, W2, b2, al2, ar2):
    src = edge_index[0]
    dst = edge_index[1]

    dst_sorted, src_sorted = lax.sort((dst, src), num_keys=1)
    counts = jax.ops.segment_sum(
        jnp.ones_like(dst), dst, num_segments=_NPAD + 24)
    offs = jnp.concatenate(
        [jnp.zeros((1,), jnp.int32),
         jnp.cumsum(counts).astype(jnp.int32)])[:_NPAD + 24]
    ssrc = jnp.concatenate([src_sorted, jnp.zeros((16,), jnp.int32)])

    x = jnp.pad(feature, ((0, _NPAD - _N), (0, 0)))

    featx1, er1 = _mm_attn(x, W1, al1, ar1, _H, 128, block_rows=512)
    h1 = _gat_edge_sc(featx1, er1, ssrc, offs, b1, 128, mean_heads=False)

    featx2, er2 = _mm_attn(h1, W2, al2, ar2, _H, 64, block_rows=512)
    out = _gat_edge_sc(featx2, er2, ssrc, offs, b2, 64, mean_heads=True)
    return out[:_N]
